# baseline (speedup 1.0000x reference)
import functools

import numpy as np
import jax
import jax.numpy as jnp
from jax import lax
from jax.experimental import pallas as pl
from jax.experimental.pallas import tpu as pltpu
from jax.experimental.pallas import tpu_sc as plsc

_GRID = [(i, j) for i in range(7) for j in range(7) if (2 <= i <= 4) or (2 <= j <= 4)]
_POS2IDX = {p: k for k, p in enumerate(_GRID)}
_MOVES = [(-1, 0), (1, 0), (0, -1), (0, 1)]
_POS = np.repeat(np.arange(33), 4)
_MOV = np.tile(np.arange(4), 33)
_MIDR = np.array([
    _POS2IDX.get((_GRID[_POS[a]][0] + _MOVES[_MOV[a]][0],
                  _GRID[_POS[a]][1] + _MOVES[_MOV[a]][1]), -1) for a in range(132)])
_TGTR = np.array([
    _POS2IDX.get((_GRID[_POS[a]][0] + 2 * _MOVES[_MOV[a]][0],
                  _GRID[_POS[a]][1] + 2 * _MOVES[_MOV[a]][1]), -1) for a in range(132)])
_OOBT = (_MIDR < 0) | (_TGTR < 0)
_MID = np.clip(_MIDR, 0, None)
_TGT = np.clip(_TGTR, 0, None)
_INB = [a for a in range(132) if not _OOBT[a]]
_OOBA = [a for a in range(132) if _OOBT[a]]
_INB_BY_TGT = sorted(_INB, key=lambda a: (_TGT[a], a))

_N = 65536
_NW = 32
_PER_W = _N // _NW
_CH = 128
_NCHUNK = _PER_W // _CH
_NGRP = _CH // 16

_mesh = plsc.VectorSubcoreMesh(core_axis_name="c", subcore_axis_name="s")

_TBL_TYPES = [
    pltpu.VMEM((160,), jnp.int32),
    pltpu.VMEM((160,), jnp.int32),
    pltpu.VMEM((160,), jnp.int32),
    pltpu.VMEM((160,), jnp.float32),
]
_SEM_TYPES = [
    pltpu.SemaphoreType.DMA,
    pltpu.SemaphoreType.DMA,
    pltpu.SemaphoreType.DMA,
    pltpu.SemaphoreType.DMA,
]
_IN_TYPES = [
    pltpu.VMEM((_CH,), jnp.int32),
    pltpu.VMEM((_CH,), jnp.int32),
    pltpu.VMEM((_CH,), jnp.float32),
    pltpu.VMEM((33, _CH), jnp.float32),
]


def _phase1(av, nv, dv, pv, tpos_v, tmid_v, ttgt_v, tnoob_v, l0, iota,
            zero16, one16):
    lane = l0 + iota
    a = av[pl.ds(l0, 16)]
    donef = dv[pl.ds(l0, 16)]
    npg = nv[pl.ds(l0, 16)]
    pos = plsc.load_gather(tpos_v, [a])
    mid = plsc.load_gather(tmid_v, [a])
    tgt = plsc.load_gather(ttgt_v, [a])
    noob = plsc.load_gather(tnoob_v, [a])
    pp = plsc.load_gather(pv, [pos, lane])
    pm = plsc.load_gather(pv, [mid, lane])
    pt = plsc.load_gather(pv, [tgt, lane])
    dof = noob * pp * pm * (1.0 - pt) * (1.0 - donef)
    do = dof > 0.0
    plsc.store_scatter(pv, [pos, lane], zero16, mask=do)
    plsc.store_scatter(pv, [mid, lane], zero16, mask=do)
    plsc.store_scatter(pv, [tgt, lane], one16, mask=do)
    n2 = npg - do.astype(jnp.int32)
    return donef, dof, n2


def _issue_in(hbms, b, bufs, ci, base_w):
    actions_h, pegs_h, npegs_h, done_h = hbms
    av, nv, dv, pv = bufs[b][0], bufs[b][1], bufs[b][2], bufs[b][3]
    sem = bufs[b][-2]
    base = base_w + ci * _CH
    pltpu.async_copy(actions_h.at[pl.ds(base, _CH)], av, sem)
    pltpu.async_copy(npegs_h.at[pl.ds(base, _CH)], nv, sem)
    pltpu.async_copy(done_h.at[pl.ds(base, _CH)], dv, sem)
    pltpu.async_copy(pegs_h.at[:, pl.ds(base, _CH)], pv, sem)


def _wait_in(hbms, b, bufs):
    actions_h, pegs_h, npegs_h, done_h = hbms
    av, nv, dv, pv = bufs[b][0], bufs[b][1], bufs[b][2], bufs[b][3]
    sem = bufs[b][-2]
    pltpu.make_async_copy(actions_h.at[pl.ds(0, _CH)], av, sem).wait()
    pltpu.make_async_copy(npegs_h.at[pl.ds(0, _CH)], nv, sem).wait()
    pltpu.make_async_copy(done_h.at[pl.ds(0, _CH)], dv, sem).wait()
    pltpu.make_async_copy(pegs_h.at[:, pl.ds(0, _CH)], pv, sem).wait()


def _pipeline(hbms, bufs, base_w, compute, issue_out, wait_out):
    _issue_in(hbms, 0, bufs, 0, base_w)

    def do_pair(pi, _):
        for b in (0, 1):
            ci = 2 * pi + b
            _wait_in(hbms, b, bufs)
            if b == 0:
                _issue_in(hbms, 1, bufs, ci + 1, base_w)
            else:
                @pl.when(pi < _NCHUNK // 2 - 1)
                def _next():
                    _issue_in(hbms, 0, bufs, ci + 1, base_w)
            @pl.when(pi > 0)
            def _drain():
                wait_out(b)
            compute(b)
            issue_out(ci, b)
        return 0

    lax.fori_loop(0, _NCHUNK // 2, do_pair, 0)
    wait_out(0)
    wait_out(1)


@functools.partial(
    pl.kernel,
    out_type=[
        jax.ShapeDtypeStruct((7, 3, 7, _N), jnp.float32),
    ],
    mesh=_mesh,
    compiler_params=pltpu.CompilerParams(
        use_tc_tiling_on_sc=False, needs_layout_passes=False),
    scratch_types=(
        _IN_TYPES + [pltpu.VMEM((7, 3, 7, _CH), jnp.float32)]
        + _IN_TYPES + [pltpu.VMEM((7, 3, 7, _CH), jnp.float32)]
        + _TBL_TYPES + _SEM_TYPES),
)
def _state_step(actions_h, pegs_h, npegs_h, done_h,
                tpos_h, tmid_h, ttgt_h, tnoob_h,
                states_h,
                a0, n0, d0, p0, s0, a1, n1, d1, p1, s1,
                tpos_v, tmid_v, ttgt_v, tnoob_v,
                si0, si1, so0, so1):
    c = lax.axis_index("c")
    s = lax.axis_index("s")
    base_w = (s * 2 + c) * _PER_W
    hbms = (actions_h, pegs_h, npegs_h, done_h)
    bufs = [(a0, n0, d0, p0, s0, si0, so0),
            (a1, n1, d1, p1, s1, si1, so1)]
    pltpu.sync_copy(tpos_h, tpos_v)
    pltpu.sync_copy(tmid_h, tmid_v)
    pltpu.sync_copy(ttgt_h, ttgt_v)
    pltpu.sync_copy(tnoob_h, tnoob_v)
    iota = lax.iota(jnp.int32, 16)
    zero16 = jnp.zeros((16,), jnp.float32)
    one16 = jnp.ones((16,), jnp.float32)

    def compute(b):
        pv, sv = bufs[b][3], bufs[b][4]

        def do_group(g, _):
            l0 = g * 16
            donef, dof, n2 = _phase1(
                bufs[b][0], bufs[b][1], bufs[b][2], pv,
                tpos_v, tmid_v, ttgt_v, tnoob_v, l0, iota, zero16, one16)
            n2f = n2.astype(jnp.float32)
            pr = (n2f - 1.0) / 31.0
            rr = (32.0 - n2f) / 31.0
            for k in range(33):
                i, j = _GRID[k]
                sv[i, 0, j, pl.ds(l0, 16)] = pv[k, pl.ds(l0, 16)]
            for i in range(7):
                for j in range(7):
                    if (i, j) not in _POS2IDX:
                        sv[i, 0, j, pl.ds(l0, 16)] = zero16
                    sv[i, 1, j, pl.ds(l0, 16)] = pr
                    sv[i, 2, j, pl.ds(l0, 16)] = rr
            return 0

        lax.fori_loop(0, _NGRP, do_group, 0)

    def issue_out(ci, b):
        sv, sem = bufs[b][4], bufs[b][-1]
        base = base_w + ci * _CH
        pltpu.async_copy(sv, states_h.at[:, :, :, pl.ds(base, _CH)], sem)

    def wait_out(b):
        sv, sem = bufs[b][4], bufs[b][-1]
        pltpu.make_async_copy(sv, states_h.at[:, :, :, pl.ds(0, _CH)], sem).wait()

    _pipeline(hbms, bufs, base_w, compute, issue_out, wait_out)


@functools.partial(
    pl.kernel,
    out_type=[
        jax.ShapeDtypeStruct((_N,), jnp.float32),
        jax.ShapeDtypeStruct((_N,), jnp.float32),
        jax.ShapeDtypeStruct((132, _N), jnp.float32),
    ],
    mesh=_mesh,
    compiler_params=pltpu.CompilerParams(
        use_tc_tiling_on_sc=False, needs_layout_passes=False),
    scratch_types=(
        _IN_TYPES + [pltpu.VMEM((_CH,), jnp.float32),
                     pltpu.VMEM((_CH,), jnp.float32),
                     pltpu.VMEM((132, _CH), jnp.float32)]
        + _IN_TYPES + [pltpu.VMEM((_CH,), jnp.float32),
                       pltpu.VMEM((_CH,), jnp.float32),
                       pltpu.VMEM((132, _CH), jnp.float32)]
        + _TBL_TYPES + _SEM_TYPES),
)
def _feas_step(actions_h, pegs_h, npegs_h, done_h,
               tpos_h, tmid_h, ttgt_h, tnoob_h,
               rew_h, nd_h, feas_h,
               a0, n0, d0, p0, r0, nd0, f0,
               a1, n1, d1, p1, r1, nd1, f1,
               tpos_v, tmid_v, ttgt_v, tnoob_v,
               si0, si1, so0, so1):
    c = lax.axis_index("c")
    s = lax.axis_index("s")
    base_w = (s * 2 + c) * _PER_W
    hbms = (actions_h, pegs_h, npegs_h, done_h)
    bufs = [(a0, n0, d0, p0, r0, nd0, f0, si0, so0),
            (a1, n1, d1, p1, r1, nd1, f1, si1, so1)]
    pltpu.sync_copy(tpos_h, tpos_v)
    pltpu.sync_copy(tmid_h, tmid_v)
    pltpu.sync_copy(ttgt_h, ttgt_v)
    pltpu.sync_copy(tnoob_h, tnoob_v)
    iota = lax.iota(jnp.int32, 16)
    zero16 = jnp.zeros((16,), jnp.float32)
    one16 = jnp.ones((16,), jnp.float32)

    def zero_oob(g, _):
        l0 = g * 16
        for aa in _OOBA:
            f0[aa, pl.ds(l0, 16)] = zero16
            f1[aa, pl.ds(l0, 16)] = zero16
        return 0
    lax.fori_loop(0, _NGRP, zero_oob, 0)

    def compute(b):
        pv, rv, ndv, fv = bufs[b][3], bufs[b][4], bufs[b][5], bufs[b][6]

        def do_group(g, _):
            l0 = g * 16
            donef, dof, n2 = _phase1(
                bufs[b][0], bufs[b][1], bufs[b][2], pv,
                tpos_v, tmid_v, ttgt_v, tnoob_v, l0, iota, zero16, one16)
            p = [pv[k, pl.ds(l0, 16)] for k in range(33)]
            sums = [zero16, zero16, zero16, zero16]
            q = None
            q_for = -1
            for t, aa in enumerate(_INB_BY_TGT):
                if _TGT[aa] != q_for:
                    q_for = _TGT[aa]
                    q = 1.0 - p[q_for]
                v = p[_POS[aa]] * p[_MID[aa]]
                v = v * q
                fv[aa, pl.ds(l0, 16)] = v
                sums[t % 4] = sums[t % 4] + v
            sumv = (sums[0] + sums[1]) + (sums[2] + sums[3])
            done_b = donef > 0.0
            nd_b = (sumv == 0.0) | (n2 == 1) | done_b
            @pl.when(jnp.any(nd_b))
            def _fixup():
                f = jnp.where(nd_b, 0.0, 1.0)
                for aa in _INB:
                    fv[aa, pl.ds(l0, 16)] = fv[aa, pl.ds(l0, 16)] * f
            win = nd_b & (~done_b) & (n2 == 1)
            rv[pl.ds(l0, 16)] = dof * (1.0 / 31.0) + jnp.where(win, 1.0, 0.0)
            ndv[pl.ds(l0, 16)] = jnp.where(nd_b, 1.0, 0.0)
            return 0

        lax.fori_loop(0, _NGRP, do_group, 0)

    def issue_out(ci, b):
        rv, ndv, fv, sem = bufs[b][4], bufs[b][5], bufs[b][6], bufs[b][-1]
        base = base_w + ci * _CH
        pltpu.async_copy(rv, rew_h.at[pl.ds(base, _CH)], sem)
        pltpu.async_copy(ndv, nd_h.at[pl.ds(base, _CH)], sem)
        pltpu.async_copy(fv, feas_h.at[:, pl.ds(base, _CH)], sem)

    def wait_out(b):
        rv, ndv, fv, sem = bufs[b][4], bufs[b][5], bufs[b][6], bufs[b][-1]
        pltpu.make_async_copy(rv, rew_h.at[pl.ds(0, _CH)], sem).wait()
        pltpu.make_async_copy(ndv, nd_h.at[pl.ds(0, _CH)], sem).wait()
        pltpu.make_async_copy(fv, feas_h.at[:, pl.ds(0, _CH)], sem).wait()

    _pipeline(hbms, bufs, base_w, compute, issue_out, wait_out)


def kernel(actions, pegs, n_pegs, done):
    donef = done.astype(jnp.float32)
    pegs_t = pegs.T
    tpos = jnp.asarray(np.pad(_POS, (0, 28)).astype(np.int32))
    tmid = jnp.asarray(np.pad(_MID, (0, 28)).astype(np.int32))
    ttgt = jnp.asarray(np.pad(_TGT, (0, 28)).astype(np.int32))
    tnoob = jnp.asarray(np.pad((~_OOBT).astype(np.float32), (0, 28)))
    (states_t,) = _state_step(actions, pegs_t, n_pegs, donef,
                              tpos, tmid, ttgt, tnoob)
    rew, ndf, feas_t = _feas_step(actions, pegs_t, n_pegs, donef,
                                  tpos, tmid, ttgt, tnoob)
    states = jnp.transpose(states_t, (3, 0, 2, 1))
    return (rew, states, ndf > 0.5, feas_t.T)


# speedup vs baseline: 1.0709x; 1.0709x over previous
import functools

import numpy as np
import jax
import jax.numpy as jnp
from jax import lax
from jax.experimental import pallas as pl
from jax.experimental.pallas import tpu as pltpu
from jax.experimental.pallas import tpu_sc as plsc

_GRID = [(i, j) for i in range(7) for j in range(7) if (2 <= i <= 4) or (2 <= j <= 4)]
_POS2IDX = {p: k for k, p in enumerate(_GRID)}
_MOVES = [(-1, 0), (1, 0), (0, -1), (0, 1)]
_POS = np.repeat(np.arange(33), 4)
_MOV = np.tile(np.arange(4), 33)
_MIDR = np.array([
    _POS2IDX.get((_GRID[_POS[a]][0] + _MOVES[_MOV[a]][0],
                  _GRID[_POS[a]][1] + _MOVES[_MOV[a]][1]), -1) for a in range(132)])
_TGTR = np.array([
    _POS2IDX.get((_GRID[_POS[a]][0] + 2 * _MOVES[_MOV[a]][0],
                  _GRID[_POS[a]][1] + 2 * _MOVES[_MOV[a]][1]), -1) for a in range(132)])
_OOBT = (_MIDR < 0) | (_TGTR < 0)
_MID = np.clip(_MIDR, 0, None)
_TGT = np.clip(_TGTR, 0, None)
_INB = [a for a in range(132) if not _OOBT[a]]
_OOBA = [a for a in range(132) if _OOBT[a]]
_INB_BY_TGT = sorted(_INB, key=lambda a: (_TGT[a], a))

_N = 65536
_NW = 32
_PER_W = _N // _NW
_CH = 128
_NCHUNK = _PER_W // _CH
_NGRP = _CH // 16

_mesh = plsc.VectorSubcoreMesh(core_axis_name="c", subcore_axis_name="s")


def _buf_types():
    return [
        pltpu.VMEM((_CH,), jnp.int32),
        pltpu.VMEM((_CH,), jnp.int32),
        pltpu.VMEM((_CH,), jnp.float32),
        pltpu.VMEM((33, _CH), jnp.float32),
        pltpu.VMEM((_CH,), jnp.float32),
        pltpu.VMEM((_CH,), jnp.float32),
        pltpu.VMEM((132, _CH), jnp.float32),
        pltpu.VMEM((7, 3, 7, _CH), jnp.float32),
    ]


@functools.partial(
    pl.kernel,
    out_type=[
        jax.ShapeDtypeStruct((_N,), jnp.float32),
        jax.ShapeDtypeStruct((7, 3, 7, _N), jnp.float32),
        jax.ShapeDtypeStruct((_N,), jnp.float32),
        jax.ShapeDtypeStruct((132, _N), jnp.float32),
    ],
    mesh=_mesh,
    compiler_params=pltpu.CompilerParams(
        use_tc_tiling_on_sc=False, needs_layout_passes=False),
    scratch_types=_buf_types() + _buf_types() + [
        pltpu.VMEM((160,), jnp.int32),
        pltpu.VMEM((160,), jnp.int32),
        pltpu.VMEM((160,), jnp.int32),
        pltpu.VMEM((160,), jnp.float32),
        pltpu.SemaphoreType.DMA,
        pltpu.SemaphoreType.DMA,
        pltpu.SemaphoreType.DMA,
        pltpu.SemaphoreType.DMA,
    ],
)
def _env_step(actions_h, pegs_h, npegs_h, done_h, tpos_h, tmid_h, ttgt_h, tnoob_h,
              rew_h, states_h, nd_h, feas_h,
              a0, n0, d0, p0, r0, nd0, f0, s0,
              a1, n1, d1, p1, r1, nd1, f1, s1,
              tpos_v, tmid_v, ttgt_v, tnoob_v,
              si0, si1, so0, so1):
    c = lax.axis_index("c")
    s = lax.axis_index("s")
    wid = s * 2 + c
    base_w = wid * _PER_W
    bufs = [(a0, n0, d0, p0, r0, nd0, f0, s0, si0, so0),
            (a1, n1, d1, p1, r1, nd1, f1, s1, si1, so1)]
    pltpu.sync_copy(tpos_h, tpos_v)
    pltpu.sync_copy(tmid_h, tmid_v)
    pltpu.sync_copy(ttgt_h, ttgt_v)
    pltpu.sync_copy(tnoob_h, tnoob_v)
    iota = lax.iota(jnp.int32, 16)
    zero16 = jnp.zeros((16,), jnp.float32)
    one16 = jnp.ones((16,), jnp.float32)

    def zero_oob(g, _):
        l0 = g * 16
        for aa in _OOBA:
            f0[aa, pl.ds(l0, 16)] = zero16
            f1[aa, pl.ds(l0, 16)] = zero16
        return 0
    lax.fori_loop(0, _NGRP, zero_oob, 0)

    def issue_in(ci, b):
        av, nv, dv, pv = bufs[b][0], bufs[b][1], bufs[b][2], bufs[b][3]
        sem = bufs[b][8]
        base = base_w + ci * _CH
        pltpu.async_copy(actions_h.at[pl.ds(base, _CH)], av, sem)
        pltpu.async_copy(npegs_h.at[pl.ds(base, _CH)], nv, sem)
        pltpu.async_copy(done_h.at[pl.ds(base, _CH)], dv, sem)
        pltpu.async_copy(pegs_h.at[:, pl.ds(base, _CH)], pv, sem)

    def wait_in(b):
        av, nv, dv, pv = bufs[b][0], bufs[b][1], bufs[b][2], bufs[b][3]
        sem = bufs[b][8]
        pltpu.make_async_copy(actions_h.at[pl.ds(0, _CH)], av, sem).wait()
        pltpu.make_async_copy(npegs_h.at[pl.ds(0, _CH)], nv, sem).wait()
        pltpu.make_async_copy(done_h.at[pl.ds(0, _CH)], dv, sem).wait()
        pltpu.make_async_copy(pegs_h.at[:, pl.ds(0, _CH)], pv, sem).wait()

    def issue_out(ci, b):
        rv, ndv, fv, sv = bufs[b][4], bufs[b][5], bufs[b][6], bufs[b][7]
        sem = bufs[b][9]
        base = base_w + ci * _CH
        pltpu.async_copy(rv, rew_h.at[pl.ds(base, _CH)], sem)
        pltpu.async_copy(ndv, nd_h.at[pl.ds(base, _CH)], sem)
        pltpu.async_copy(fv, feas_h.at[:, pl.ds(base, _CH)], sem)
        pltpu.async_copy(sv, states_h.at[:, :, :, pl.ds(base, _CH)], sem)

    def wait_out(b):
        rv, ndv, fv, sv = bufs[b][4], bufs[b][5], bufs[b][6], bufs[b][7]
        sem = bufs[b][9]
        pltpu.make_async_copy(rv, rew_h.at[pl.ds(0, _CH)], sem).wait()
        pltpu.make_async_copy(ndv, nd_h.at[pl.ds(0, _CH)], sem).wait()
        pltpu.make_async_copy(fv, feas_h.at[:, pl.ds(0, _CH)], sem).wait()
        pltpu.make_async_copy(sv, states_h.at[:, :, :, pl.ds(0, _CH)], sem).wait()

    def compute(b):
        av, nv, dv, pv, rv, ndv, fv, sv = bufs[b][:8]

        def do_group(g, _):
            l0 = g * 16
            lane = l0 + iota
            a = av[pl.ds(l0, 16)]
            donef = dv[pl.ds(l0, 16)]
            npg = nv[pl.ds(l0, 16)]
            pos = plsc.load_gather(tpos_v, [a])
            mid = plsc.load_gather(tmid_v, [a])
            tgt = plsc.load_gather(ttgt_v, [a])
            noob = plsc.load_gather(tnoob_v, [a])
            pp = plsc.load_gather(pv, [pos, lane])
            pm = plsc.load_gather(pv, [mid, lane])
            pt = plsc.load_gather(pv, [tgt, lane])
            nd = 1.0 - donef
            dof = noob * pp * pm * (1.0 - pt) * nd
            do = dof > 0.0
            plsc.store_scatter(pv, [pos, lane], zero16, mask=do)
            plsc.store_scatter(pv, [mid, lane], zero16, mask=do)
            plsc.store_scatter(pv, [tgt, lane], one16, mask=do)
            n2 = npg - do.astype(jnp.int32)
            p = [pv[k, pl.ds(l0, 16)] for k in range(33)]
            sums = [zero16, zero16, zero16, zero16]
            q = None
            q_for = -1
            for t, aa in enumerate(_INB_BY_TGT):
                if _TGT[aa] != q_for:
                    q_for = _TGT[aa]
                    q = 1.0 - p[q_for]
                v = p[_POS[aa]] * p[_MID[aa]]
                v = v * q
                fv[aa, pl.ds(l0, 16)] = v
                sums[t % 4] = sums[t % 4] + v
            sumv = (sums[0] + sums[1]) + (sums[2] + sums[3])
            done_b = donef > 0.0
            nd_b = (sumv == 0.0) | (n2 == 1) | done_b
            @pl.when(jnp.any(nd_b))
            def _fixup():
                f = jnp.where(nd_b, 0.0, 1.0)
                for aa in _INB:
                    fv[aa, pl.ds(l0, 16)] = fv[aa, pl.ds(l0, 16)] * f
            n2f = n2.astype(jnp.float32)
            win = nd_b & (~done_b) & (n2 == 1)
            rew = dof * (1.0 / 31.0) + jnp.where(win, 1.0, 0.0)
            rv[pl.ds(l0, 16)] = rew
            ndv[pl.ds(l0, 16)] = jnp.where(nd_b, 1.0, 0.0)
            pr = (n2f - 1.0) / 31.0
            rr = (32.0 - n2f) / 31.0
            for k in range(33):
                i, j = _GRID[k]
                sv[i, 0, j, pl.ds(l0, 16)] = p[k]
            for i in range(7):
                for j in range(7):
                    if (i, j) not in _POS2IDX:
                        sv[i, 0, j, pl.ds(l0, 16)] = zero16
                    sv[i, 1, j, pl.ds(l0, 16)] = pr
                    sv[i, 2, j, pl.ds(l0, 16)] = rr
            return 0

        lax.fori_loop(0, _NGRP, do_group, 0)

    issue_in(0, 0)

    def do_pair(pi, _):
        for b in (0, 1):
            ci = 2 * pi + b
            wait_in(b)
            if b == 0:
                issue_in(ci + 1, 1)
            else:
                @pl.when(pi < _NCHUNK // 2 - 1)
                def _next():
                    issue_in(ci + 1, 0)
            @pl.when(pi > 0)
            def _drain():
                wait_out(b)
            compute(b)
            issue_out(ci, b)
        return 0

    lax.fori_loop(0, _NCHUNK // 2, do_pair, 0)
    wait_out(0)
    wait_out(1)


def kernel(actions, pegs, n_pegs, done):
    donef = done.astype(jnp.float32)
    tpos = jnp.asarray(np.pad(_POS, (0, 28)).astype(np.int32))
    tmid = jnp.asarray(np.pad(_MID, (0, 28)).astype(np.int32))
    ttgt = jnp.asarray(np.pad(_TGT, (0, 28)).astype(np.int32))
    tnoob = jnp.asarray(np.pad((~_OOBT).astype(np.float32), (0, 28)))
    rew, states_t, ndf, feas_t = _env_step(
        actions, pegs.T, n_pegs, donef, tpos, tmid, ttgt, tnoob)
    states = jnp.transpose(states_t, (3, 0, 2, 1))
    return (rew, states, ndf > 0.5, feas_t.T)


# speedup vs baseline: 1.0733x; 1.0023x over previous
import functools

import numpy as np
import jax
import jax.numpy as jnp
from jax import lax
from jax.experimental import pallas as pl
from jax.experimental.pallas import tpu as pltpu
from jax.experimental.pallas import tpu_sc as plsc

_GRID = [(i, j) for i in range(7) for j in range(7) if (2 <= i <= 4) or (2 <= j <= 4)]
_POS2IDX = {p: k for k, p in enumerate(_GRID)}
_MOVES = [(-1, 0), (1, 0), (0, -1), (0, 1)]
_POS = np.repeat(np.arange(33), 4)
_MOV = np.tile(np.arange(4), 33)
_MIDR = np.array([
    _POS2IDX.get((_GRID[_POS[a]][0] + _MOVES[_MOV[a]][0],
                  _GRID[_POS[a]][1] + _MOVES[_MOV[a]][1]), -1) for a in range(132)])
_TGTR = np.array([
    _POS2IDX.get((_GRID[_POS[a]][0] + 2 * _MOVES[_MOV[a]][0],
                  _GRID[_POS[a]][1] + 2 * _MOVES[_MOV[a]][1]), -1) for a in range(132)])
_OOBT = (_MIDR < 0) | (_TGTR < 0)
_MID = np.clip(_MIDR, 0, None)
_TGT = np.clip(_TGTR, 0, None)
_INB = [a for a in range(132) if not _OOBT[a]]
_OOBA = [a for a in range(132) if _OOBT[a]]
_INB_BY_TGT = sorted(_INB, key=lambda a: (_TGT[a], a))

_N = 65536
_NW = 32
_PER_W = _N // _NW
_CH = 128
_NCHUNK = _PER_W // _CH
_NGRP = _CH // 16

_mesh = plsc.VectorSubcoreMesh(core_axis_name="c", subcore_axis_name="s")


def _buf_types():
    return [
        pltpu.VMEM((_CH,), jnp.int32),
        pltpu.VMEM((_CH,), jnp.int32),
        pltpu.VMEM((_CH,), jnp.float32),
        pltpu.VMEM((33, _CH), jnp.float32),
        pltpu.VMEM((_CH,), jnp.float32),
        pltpu.VMEM((_CH,), jnp.float32),
        pltpu.VMEM((132, _CH), jnp.float32),
        pltpu.VMEM((7, 3, 7, _CH), jnp.float32),
    ]


@functools.partial(
    pl.kernel,
    out_type=[
        jax.ShapeDtypeStruct((_N,), jnp.float32),
        jax.ShapeDtypeStruct((7, 3, 7, _N), jnp.float32),
        jax.ShapeDtypeStruct((_N,), jnp.float32),
        jax.ShapeDtypeStruct((132, _N), jnp.float32),
    ],
    mesh=_mesh,
    compiler_params=pltpu.CompilerParams(
        use_tc_tiling_on_sc=False, needs_layout_passes=False),
    scratch_types=_buf_types() + _buf_types() + [
        pltpu.VMEM((160,), jnp.int32),
        pltpu.VMEM((160,), jnp.int32),
        pltpu.VMEM((160,), jnp.int32),
        pltpu.VMEM((160,), jnp.float32),
        pltpu.SemaphoreType.DMA,
        pltpu.SemaphoreType.DMA,
        pltpu.SemaphoreType.DMA,
        pltpu.SemaphoreType.DMA,
    ],
)
def _env_step(actions_h, pegs_h, npegs_h, done_h, tpos_h, tmid_h, ttgt_h, tnoob_h,
              rew_h, states_h, nd_h, feas_h,
              a0, n0, d0, p0, r0, nd0, f0, s0,
              a1, n1, d1, p1, r1, nd1, f1, s1,
              tpos_v, tmid_v, ttgt_v, tnoob_v,
              si0, si1, so0, so1):
    c = lax.axis_index("c")
    s = lax.axis_index("s")
    wid = s * 2 + c
    base_w = wid * _PER_W
    bufs = [(a0, n0, d0, p0, r0, nd0, f0, s0, si0, so0),
            (a1, n1, d1, p1, r1, nd1, f1, s1, si1, so1)]
    pltpu.sync_copy(tpos_h, tpos_v)
    pltpu.sync_copy(tmid_h, tmid_v)
    pltpu.sync_copy(ttgt_h, ttgt_v)
    pltpu.sync_copy(tnoob_h, tnoob_v)
    iota = lax.iota(jnp.int32, 16)
    zero16 = jnp.zeros((16,), jnp.float32)
    one16 = jnp.ones((16,), jnp.float32)

    def zero_oob(g, _):
        l0 = g * 16
        for aa in _OOBA:
            f0[aa, pl.ds(l0, 16)] = zero16
            f1[aa, pl.ds(l0, 16)] = zero16
        return 0
    lax.fori_loop(0, _NGRP, zero_oob, 0)

    def issue_in(ci, b):
        av, nv, dv, pv = bufs[b][0], bufs[b][1], bufs[b][2], bufs[b][3]
        sem = bufs[b][8]
        base = base_w + ci * _CH
        pltpu.async_copy(actions_h.at[pl.ds(base, _CH)], av, sem)
        pltpu.async_copy(npegs_h.at[pl.ds(base, _CH)], nv, sem)
        pltpu.async_copy(done_h.at[pl.ds(base, _CH)], dv, sem)
        pltpu.async_copy(pegs_h.at[:, pl.ds(base, _CH)], pv, sem)

    def wait_in(b):
        av, nv, dv, pv = bufs[b][0], bufs[b][1], bufs[b][2], bufs[b][3]
        sem = bufs[b][8]
        pltpu.make_async_copy(actions_h.at[pl.ds(0, _CH)], av, sem).wait()
        pltpu.make_async_copy(npegs_h.at[pl.ds(0, _CH)], nv, sem).wait()
        pltpu.make_async_copy(done_h.at[pl.ds(0, _CH)], dv, sem).wait()
        pltpu.make_async_copy(pegs_h.at[:, pl.ds(0, _CH)], pv, sem).wait()

    def issue_out(ci, b):
        rv, ndv, fv, sv = bufs[b][4], bufs[b][5], bufs[b][6], bufs[b][7]
        sem = bufs[b][9]
        base = base_w + ci * _CH
        pltpu.async_copy(rv, rew_h.at[pl.ds(base, _CH)], sem)
        pltpu.async_copy(ndv, nd_h.at[pl.ds(base, _CH)], sem)
        pltpu.async_copy(fv, feas_h.at[:, pl.ds(base, _CH)], sem)
        pltpu.async_copy(sv, states_h.at[:, :, :, pl.ds(base, _CH)], sem)

    def wait_out(b):
        rv, ndv, fv, sv = bufs[b][4], bufs[b][5], bufs[b][6], bufs[b][7]
        sem = bufs[b][9]
        pltpu.make_async_copy(rv, rew_h.at[pl.ds(0, _CH)], sem).wait()
        pltpu.make_async_copy(ndv, nd_h.at[pl.ds(0, _CH)], sem).wait()
        pltpu.make_async_copy(fv, feas_h.at[:, pl.ds(0, _CH)], sem).wait()
        pltpu.make_async_copy(sv, states_h.at[:, :, :, pl.ds(0, _CH)], sem).wait()

    def compute(b):
        av, nv, dv, pv, rv, ndv, fv, sv = bufs[b][:8]

        @plsc.parallel_loop(0, _CH, step=16)
        def do_group(l0):
            lane = l0 + iota
            a = av[pl.ds(l0, 16)]
            donef = dv[pl.ds(l0, 16)]
            npg = nv[pl.ds(l0, 16)]
            pos = plsc.load_gather(tpos_v, [a])
            mid = plsc.load_gather(tmid_v, [a])
            tgt = plsc.load_gather(ttgt_v, [a])
            noob = plsc.load_gather(tnoob_v, [a])
            pp = plsc.load_gather(pv, [pos, lane])
            pm = plsc.load_gather(pv, [mid, lane])
            pt = plsc.load_gather(pv, [tgt, lane])
            nd = 1.0 - donef
            dof = noob * pp * pm * (1.0 - pt) * nd
            do = dof > 0.0
            plsc.store_scatter(pv, [pos, lane], zero16, mask=do)
            plsc.store_scatter(pv, [mid, lane], zero16, mask=do)
            plsc.store_scatter(pv, [tgt, lane], one16, mask=do)
            n2 = npg - do.astype(jnp.int32)
            p = [pv[k, pl.ds(l0, 16)] for k in range(33)]
            sums = [zero16, zero16, zero16, zero16]
            q = None
            q_for = -1
            for t, aa in enumerate(_INB_BY_TGT):
                if _TGT[aa] != q_for:
                    q_for = _TGT[aa]
                    q = 1.0 - p[q_for]
                v = p[_POS[aa]] * p[_MID[aa]]
                v = v * q
                fv[aa, pl.ds(l0, 16)] = v
                sums[t % 4] = sums[t % 4] + v
            sumv = (sums[0] + sums[1]) + (sums[2] + sums[3])
            done_b = donef > 0.0
            nd_b = (sumv == 0.0) | (n2 == 1) | done_b
            @pl.when(jnp.any(nd_b))
            def _fixup():
                f = jnp.where(nd_b, 0.0, 1.0)
                for aa in _INB:
                    fv[aa, pl.ds(l0, 16)] = fv[aa, pl.ds(l0, 16)] * f
            n2f = n2.astype(jnp.float32)
            win = nd_b & (~done_b) & (n2 == 1)
            rew = dof * (1.0 / 31.0) + jnp.where(win, 1.0, 0.0)
            rv[pl.ds(l0, 16)] = rew
            ndv[pl.ds(l0, 16)] = jnp.where(nd_b, 1.0, 0.0)
            pr = (n2f - 1.0) / 31.0
            rr = (32.0 - n2f) / 31.0
            for k in range(33):
                i, j = _GRID[k]
                sv[i, 0, j, pl.ds(l0, 16)] = p[k]
            for i in range(7):
                for j in range(7):
                    if (i, j) not in _POS2IDX:
                        sv[i, 0, j, pl.ds(l0, 16)] = zero16
                    sv[i, 1, j, pl.ds(l0, 16)] = pr
                    sv[i, 2, j, pl.ds(l0, 16)] = rr

    issue_in(0, 0)

    def do_pair(pi, _):
        for b in (0, 1):
            ci = 2 * pi + b
            wait_in(b)
            if b == 0:
                issue_in(ci + 1, 1)
            else:
                @pl.when(pi < _NCHUNK // 2 - 1)
                def _next():
                    issue_in(ci + 1, 0)
            @pl.when(pi > 0)
            def _drain():
                wait_out(b)
            compute(b)
            issue_out(ci, b)
        return 0

    lax.fori_loop(0, _NCHUNK // 2, do_pair, 0)
    wait_out(0)
    wait_out(1)


def kernel(actions, pegs, n_pegs, done):
    donef = done.astype(jnp.float32)
    tpos = jnp.asarray(np.pad(_POS, (0, 28)).astype(np.int32))
    tmid = jnp.asarray(np.pad(_MID, (0, 28)).astype(np.int32))
    ttgt = jnp.asarray(np.pad(_TGT, (0, 28)).astype(np.int32))
    tnoob = jnp.asarray(np.pad((~_OOBT).astype(np.float32), (0, 28)))
    rew, states_t, ndf, feas_t = _env_step(
        actions, pegs.T, n_pegs, donef, tpos, tmid, ttgt, tnoob)
    states = jnp.transpose(states_t, (3, 0, 2, 1))
    return (rew, states, ndf > 0.5, feas_t.T)


# speedup vs baseline: 1.0936x; 1.0188x over previous
import functools

import numpy as np
import jax
import jax.numpy as jnp
from jax import lax
from jax.experimental import pallas as pl
from jax.experimental.pallas import tpu as pltpu
from jax.experimental.pallas import tpu_sc as plsc

_GRID = [(i, j) for i in range(7) for j in range(7) if (2 <= i <= 4) or (2 <= j <= 4)]
_POS2IDX = {p: k for k, p in enumerate(_GRID)}
_MOVES = [(-1, 0), (1, 0), (0, -1), (0, 1)]
_POS = np.repeat(np.arange(33), 4)
_MOV = np.tile(np.arange(4), 33)
_MIDR = np.array([
    _POS2IDX.get((_GRID[_POS[a]][0] + _MOVES[_MOV[a]][0],
                  _GRID[_POS[a]][1] + _MOVES[_MOV[a]][1]), -1) for a in range(132)])
_TGTR = np.array([
    _POS2IDX.get((_GRID[_POS[a]][0] + 2 * _MOVES[_MOV[a]][0],
                  _GRID[_POS[a]][1] + 2 * _MOVES[_MOV[a]][1]), -1) for a in range(132)])
_OOBT = (_MIDR < 0) | (_TGTR < 0)
_MID = np.clip(_MIDR, 0, None)
_TGT = np.clip(_TGTR, 0, None)
_INB = [a for a in range(132) if not _OOBT[a]]
_OOBA = [a for a in range(132) if _OOBT[a]]
_INB_BY_TGT = sorted(_INB, key=lambda a: (_TGT[a], a))

_N = 65536
_NW = 32
_PER_W = _N // _NW
_CH = 128
_NCHUNK = _PER_W // _CH
_NGRP = _CH // 16

_mesh = plsc.VectorSubcoreMesh(core_axis_name="c", subcore_axis_name="s")


def _buf_types():
    return [
        pltpu.VMEM((_CH,), jnp.int32),
        pltpu.VMEM((_CH,), jnp.int32),
        pltpu.VMEM((_CH,), jnp.float32),
        pltpu.VMEM((33, _CH), jnp.float32),
        pltpu.VMEM((_CH,), jnp.float32),
        pltpu.VMEM((_CH,), jnp.float32),
        pltpu.VMEM((132, _CH), jnp.float32),
        pltpu.VMEM((7, 3, 7, _CH), jnp.float32),
    ]


@functools.partial(
    pl.kernel,
    out_type=[
        jax.ShapeDtypeStruct((_N,), jnp.float32),
        jax.ShapeDtypeStruct((7, 3, 7, _N), jnp.float32),
        jax.ShapeDtypeStruct((_N,), jnp.float32),
        jax.ShapeDtypeStruct((132, _N), jnp.float32),
    ],
    mesh=_mesh,
    compiler_params=pltpu.CompilerParams(
        use_tc_tiling_on_sc=False, needs_layout_passes=False),
    scratch_types=_buf_types() + _buf_types() + [
        pltpu.VMEM((160,), jnp.int32),
        pltpu.VMEM((160,), jnp.int32),
        pltpu.VMEM((160,), jnp.int32),
        pltpu.VMEM((160,), jnp.float32),
        pltpu.SemaphoreType.DMA,
        pltpu.SemaphoreType.DMA,
        pltpu.SemaphoreType.DMA,
        pltpu.SemaphoreType.DMA,
    ],
)
def _env_step(actions_h, pegs_h, npegs_h, done_h, tpos_h, tmid_h, ttgt_h, tnoob_h,
              rew_h, states_h, nd_h, feas_h,
              a0, n0, d0, p0, r0, nd0, f0, s0,
              a1, n1, d1, p1, r1, nd1, f1, s1,
              tpos_v, tmid_v, ttgt_v, tnoob_v,
              si0, si1, so0, so1):
    c = lax.axis_index("c")
    s = lax.axis_index("s")
    wid = s * 2 + c
    base_w = wid * _PER_W
    bufs = [(a0, n0, d0, p0, r0, nd0, f0, s0, si0, so0),
            (a1, n1, d1, p1, r1, nd1, f1, s1, si1, so1)]
    iota = lax.iota(jnp.int32, 16)
    zero16 = jnp.zeros((16,), jnp.float32)
    one16 = jnp.ones((16,), jnp.float32)

    def issue_in(ci, b):
        av, nv, dv, pv = bufs[b][0], bufs[b][1], bufs[b][2], bufs[b][3]
        sem = bufs[b][8]
        base = base_w + ci * _CH
        pltpu.async_copy(actions_h.at[pl.ds(base, _CH)], av, sem)
        pltpu.async_copy(npegs_h.at[pl.ds(base, _CH)], nv, sem)
        pltpu.async_copy(done_h.at[pl.ds(base, _CH)], dv, sem)
        pltpu.async_copy(pegs_h.at[:, pl.ds(base, _CH)], pv, sem)

    def wait_in(b):
        av, nv, dv, pv = bufs[b][0], bufs[b][1], bufs[b][2], bufs[b][3]
        sem = bufs[b][8]
        pltpu.make_async_copy(actions_h.at[pl.ds(0, _CH)], av, sem).wait()
        pltpu.make_async_copy(npegs_h.at[pl.ds(0, _CH)], nv, sem).wait()
        pltpu.make_async_copy(done_h.at[pl.ds(0, _CH)], dv, sem).wait()
        pltpu.make_async_copy(pegs_h.at[:, pl.ds(0, _CH)], pv, sem).wait()

    def issue_out(ci, b):
        rv, ndv, fv, sv = bufs[b][4], bufs[b][5], bufs[b][6], bufs[b][7]
        sem = bufs[b][9]
        base = base_w + ci * _CH
        pltpu.async_copy(rv, rew_h.at[pl.ds(base, _CH)], sem)
        pltpu.async_copy(ndv, nd_h.at[pl.ds(base, _CH)], sem)
        pltpu.async_copy(fv, feas_h.at[:, pl.ds(base, _CH)], sem)
        pltpu.async_copy(sv, states_h.at[:, :, :, pl.ds(base, _CH)], sem)

    def wait_out(b):
        rv, ndv, fv, sv = bufs[b][4], bufs[b][5], bufs[b][6], bufs[b][7]
        sem = bufs[b][9]
        pltpu.make_async_copy(rv, rew_h.at[pl.ds(0, _CH)], sem).wait()
        pltpu.make_async_copy(ndv, nd_h.at[pl.ds(0, _CH)], sem).wait()
        pltpu.make_async_copy(fv, feas_h.at[:, pl.ds(0, _CH)], sem).wait()
        pltpu.make_async_copy(sv, states_h.at[:, :, :, pl.ds(0, _CH)], sem).wait()

    def compute(b):
        av, nv, dv, pv, rv, ndv, fv, sv = bufs[b][:8]

        @plsc.parallel_loop(0, _CH, step=16)
        def do_group(l0):
            lane = l0 + iota
            a = av[pl.ds(l0, 16)]
            donef = dv[pl.ds(l0, 16)]
            npg = nv[pl.ds(l0, 16)]
            pos = plsc.load_gather(tpos_v, [a])
            mid = plsc.load_gather(tmid_v, [a])
            tgt = plsc.load_gather(ttgt_v, [a])
            noob = plsc.load_gather(tnoob_v, [a])
            pp = plsc.load_gather(pv, [pos, lane])
            pm = plsc.load_gather(pv, [mid, lane])
            pt = plsc.load_gather(pv, [tgt, lane])
            nd = 1.0 - donef
            dof = noob * pp * pm * (1.0 - pt) * nd
            do = dof > 0.0
            plsc.store_scatter(pv, [pos, lane], zero16, mask=do)
            plsc.store_scatter(pv, [mid, lane], zero16, mask=do)
            plsc.store_scatter(pv, [tgt, lane], one16, mask=do)
            n2 = npg - do.astype(jnp.int32)
            p = [pv[k, pl.ds(l0, 16)] for k in range(33)]
            sums = [zero16, zero16, zero16, zero16]
            q = None
            q_for = -1
            for t, aa in enumerate(_INB_BY_TGT):
                if _TGT[aa] != q_for:
                    q_for = _TGT[aa]
                    q = 1.0 - p[q_for]
                v = p[_POS[aa]] * p[_MID[aa]]
                v = v * q
                fv[aa, pl.ds(l0, 16)] = v
                sums[t % 4] = sums[t % 4] + v
            sumv = (sums[0] + sums[1]) + (sums[2] + sums[3])
            done_b = donef > 0.0
            nd_b = (sumv == 0.0) | (n2 == 1) | done_b
            @pl.when(jnp.any(nd_b))
            def _fixup():
                f = jnp.where(nd_b, 0.0, 1.0)
                for aa in _INB:
                    fv[aa, pl.ds(l0, 16)] = fv[aa, pl.ds(l0, 16)] * f
            n2f = n2.astype(jnp.float32)
            win = nd_b & (~done_b) & (n2 == 1)
            rew = dof * (1.0 / 31.0) + jnp.where(win, 1.0, 0.0)
            rv[pl.ds(l0, 16)] = rew
            ndv[pl.ds(l0, 16)] = jnp.where(nd_b, 1.0, 0.0)
            pr = (n2f - 1.0) / 31.0
            rr = (32.0 - n2f) / 31.0
            for k in range(33):
                i, j = _GRID[k]
                sv[i, 0, j, pl.ds(l0, 16)] = p[k]
            for i in range(7):
                for j in range(7):
                    if (i, j) not in _POS2IDX:
                        sv[i, 0, j, pl.ds(l0, 16)] = zero16
                    sv[i, 1, j, pl.ds(l0, 16)] = pr
                    sv[i, 2, j, pl.ds(l0, 16)] = rr

    issue_in(0, 0)
    pltpu.async_copy(tpos_h, tpos_v, so0)
    pltpu.async_copy(tmid_h, tmid_v, so0)
    pltpu.async_copy(ttgt_h, ttgt_v, so0)
    pltpu.async_copy(tnoob_h, tnoob_v, so0)

    @plsc.parallel_loop(0, _CH, step=16)
    def zero_oob(l0):
        for aa in _OOBA:
            f0[aa, pl.ds(l0, 16)] = zero16
            f1[aa, pl.ds(l0, 16)] = zero16

    pltpu.make_async_copy(tpos_h, tpos_v, so0).wait()
    pltpu.make_async_copy(tmid_h, tmid_v, so0).wait()
    pltpu.make_async_copy(ttgt_h, ttgt_v, so0).wait()
    pltpu.make_async_copy(tnoob_h, tnoob_v, so0).wait()

    def do_pair(pi, _):
        for b in (0, 1):
            ci = 2 * pi + b
            wait_in(b)
            if b == 0:
                issue_in(ci + 1, 1)
            else:
                @pl.when(pi < _NCHUNK // 2 - 1)
                def _next():
                    issue_in(ci + 1, 0)
            @pl.when(pi > 0)
            def _drain():
                wait_out(b)
            compute(b)
            issue_out(ci, b)
        return 0

    lax.fori_loop(0, _NCHUNK // 2, do_pair, 0)
    wait_out(0)
    wait_out(1)


def kernel(actions, pegs, n_pegs, done):
    donef = done.astype(jnp.float32)
    tpos = jnp.asarray(np.pad(_POS, (0, 28)).astype(np.int32))
    tmid = jnp.asarray(np.pad(_MID, (0, 28)).astype(np.int32))
    ttgt = jnp.asarray(np.pad(_TGT, (0, 28)).astype(np.int32))
    tnoob = jnp.asarray(np.pad((~_OOBT).astype(np.float32), (0, 28)))
    rew, states_t, ndf, feas_t = _env_step(
        actions, pegs.T, n_pegs, donef, tpos, tmid, ttgt, tnoob)
    states = jnp.transpose(states_t, (3, 0, 2, 1))
    return (rew, states, ndf > 0.5, feas_t.T)
